# baseline (speedup 1.0000x reference)
import jax
import jax.numpy as jnp
from jax import lax
from jax.experimental import pallas as pl
from jax.experimental.pallas import tpu as pltpu
from jax.experimental.pallas import tpu_sc as plsc

N = 10000
D = 128
NC = 2
NS = 16
NW = NC * NS
K = 128
RPT = 632
RPAD = NS * RPT
TRASH = N
SPLIT0 = 0.72


def _core_chunks(E):
    cht = -(-E // (NS * K))
    ch0 = 2 * round(SPLIT0 * cht / 2)
    ch0 = max(2, min(ch0, cht))
    ch1 = max(2, 2 * (-(-(cht - ch0) // 2)))
    return ch0, ch1


def _sc_agg(ch0, ch1, with_deg):
    chmax = max(ch0, ch1)
    mesh = plsc.VectorSubcoreMesh(core_axis_name="c", subcore_axis_name="s")

    out_type = [jax.ShapeDtypeStruct((NC, RPAD, D), jnp.float32)]
    scratch = [
        pltpu.VMEM((chmax, K), jnp.int32),
        pltpu.VMEM((K,), jnp.int32),
        pltpu.VMEM((K,), jnp.int32),
        pltpu.VMEM((K, D), jnp.float32),
        pltpu.VMEM((K, D), jnp.float32),
        pltpu.VMEM_SHARED((RPAD, D), jnp.float32),
        pltpu.SemaphoreType.DMA,
        pltpu.SemaphoreType.DMA,
        pltpu.SemaphoreType.DMA,
        pltpu.SemaphoreType.DMA,
    ]
    if with_deg:
        out_type.append(jax.ShapeDtypeStruct((NC * RPAD,), jnp.float32))
        scratch.append(pltpu.VMEM((K,), jnp.float32))
        scratch.append(pltpu.VMEM_SHARED((RPAD,), jnp.float32))
        scratch.append(pltpu.VMEM((RPT,), jnp.float32))

    def body(x_hbm, src_hbm, dst_hbm, z2_hbm, z1_hbm, *rest):
        if with_deg:
            (acc_out, deg_out, dst_v, srcb0, srcb1, rows0, rows1, acc_s,
             gsem0, gsem1, isem0, isem1, ones_v, deg_s, deg_stage) = rest
        else:
            (acc_out, dst_v, srcb0, srcb1, rows0, rows1, acc_s,
             gsem0, gsem1, isem0, isem1) = rest
        c = lax.axis_index("c")
        s = lax.axis_index("s")
        wid = s * NC + c
        base0 = wid * (chmax * K)
        my_chunks = jnp.where(c == 0, ch0, ch1)

        srcb = (srcb0, srcb1)
        rows = (rows0, rows1)
        gsem = (gsem0, gsem1)
        isem = (isem0, isem1)

        pltpu.sync_copy(dst_hbm.at[wid], dst_v)

        pltpu.sync_copy(z2_hbm.at[pl.ds(s * RPT, RPT)],
                        acc_s.at[pl.ds(s * RPT, RPT)])
        if with_deg:
            pltpu.sync_copy(z1_hbm.at[pl.ds(s * RPT, RPT)], deg_stage)
            pltpu.sync_copy(deg_stage, deg_s.at[pl.ds(s * RPT, RPT)])
            for j in range(K // 16):
                ones_v[pl.ds(j * 16, 16)] = jnp.ones((16,), jnp.float32)
        plsc.subcore_barrier()

        def src_start(i, b):
            pltpu.async_copy(src_hbm.at[pl.ds(base0 + i * K, K)],
                             srcb[b], isem[b])

        def src_wait(b):
            pltpu.make_async_copy(src_hbm.at[pl.ds(0, K)], srcb[b],
                                  isem[b]).wait()

        def gather_start(b):
            pltpu.async_copy(x_hbm.at[srcb[b]], rows[b], gsem[b])

        def gather_wait(b):
            pltpu.make_async_copy(x_hbm.at[srcb[b]], rows[b],
                                  gsem[b]).wait()

        def scatter(i, b):
            pltpu.sync_copy(rows[b], acc_s.at[dst_v.at[i]], add=True)
            if with_deg:
                pltpu.sync_copy(ones_v, deg_s.at[dst_v.at[i]], add=True)

        src_start(0, 0)
        src_wait(0)
        gather_start(0)
        src_start(1, 1)
        src_wait(1)
        gather_start(1)

        def pair(j, carry):
            i0 = 2 * j
            gather_wait(0)
            scatter(i0, 0)

            @pl.when(i0 + 2 < my_chunks)
            def _():
                src_start(i0 + 2, 0)
                src_wait(0)
                gather_start(0)

            gather_wait(1)
            scatter(i0 + 1, 1)

            @pl.when(i0 + 3 < my_chunks)
            def _():
                src_start(i0 + 3, 1)
                src_wait(1)
                gather_start(1)

            return carry

        lax.fori_loop(0, my_chunks // 2, pair, 0)
        plsc.subcore_barrier()

        pltpu.sync_copy(acc_s.at[pl.ds(s * RPT, RPT)],
                        acc_out.at[c, pl.ds(s * RPT, RPT)])
        if with_deg:
            pltpu.sync_copy(deg_s.at[pl.ds(s * RPT, RPT)], deg_stage)
            pltpu.sync_copy(deg_stage,
                            deg_out.at[pl.ds(c * RPAD + s * RPT, RPT)])

    return pl.kernel(body, out_type=out_type, mesh=mesh,
                     scratch_types=scratch)


def _split_edges(idx, E, ch0, ch1, fill):
    chmax = max(ch0, ch1)
    cap0 = NS * ch0 * K
    cap1 = NS * ch1 * K
    idx_p = jnp.concatenate(
        [idx, jnp.full((cap0 + cap1 - E,), fill, jnp.int32)])
    p0 = idx_p[:cap0].reshape(NS, ch0, K)
    p1 = idx_p[cap0:].reshape(NS, ch1, K)
    p0 = jnp.pad(p0, ((0, 0), (0, chmax - ch0), (0, 0)),
                 constant_values=fill)
    p1 = jnp.pad(p1, ((0, 0), (0, chmax - ch1), (0, 0)),
                 constant_values=fill)
    return jnp.stack([p0, p1], axis=1).reshape(NW, chmax, K)


def _dot_t(a, b):
    return lax.dot_general(a, b, (((1,), (1,)), ((), ())),
                           preferred_element_type=jnp.float32)


def _tc1_body(acc_ref, invd_ref, x_ref, wl_ref, bl_ref, wr_ref,
              g_ref, b_ref, out_ref):
    aggsum = acc_ref[0, :N, :] + acc_ref[1, :N, :]
    agg = aggsum * invd_ref[...]
    p = _dot_t(agg, wl_ref[...]) + bl_ref[...] + _dot_t(x_ref[...], wr_ref[...])
    mu = jnp.mean(p, axis=0, keepdims=True)
    var = jnp.mean((p - mu) ** 2, axis=0, keepdims=True)
    h = (p - mu) * lax.rsqrt(var + 1e-5) * g_ref[...] + b_ref[...]
    out_ref[...] = jnp.maximum(h, 0.0)


def _tc2_body(acc_ref, invd_ref, h_ref, wl_ref, bl_ref, wr_ref,
              g_ref, b_ref, wfc_ref, bfc_ref, out_ref):
    aggsum = acc_ref[0, :N, :] + acc_ref[1, :N, :]
    agg = aggsum * invd_ref[...]
    p = _dot_t(agg, wl_ref[...]) + bl_ref[...] + _dot_t(h_ref[...], wr_ref[...])
    mu = jnp.mean(p, axis=0, keepdims=True)
    var = jnp.mean((p - mu) ** 2, axis=0, keepdims=True)
    h2 = (p - mu) * lax.rsqrt(var + 1e-5) * g_ref[...] + b_ref[...]
    h2 = jnp.maximum(h2, 0.0)
    out_ref[...] = _dot_t(h2, wfc_ref[...]) + bfc_ref[...]


def kernel(x, edge_index, W_l1, b_l1, W_r1, bn1_g, bn1_b,
           W_l2, b_l2, W_r2, bn2_g, bn2_b, W_fc, b_fc):
    E = edge_index.shape[1]
    ch0, ch1 = _core_chunks(E)
    src_p = _split_edges(edge_index[0], E, ch0, ch1, 0).reshape(-1)
    dst_p = _split_edges(edge_index[1], E, ch0, ch1, TRASH)
    z2 = jnp.zeros((RPAD, D), jnp.float32)
    z1 = jnp.zeros((RPAD,), jnp.float32)

    acc1, degp = _sc_agg(ch0, ch1, True)(x, src_p, dst_p, z2, z1)
    deg = degp[:N] + degp[RPAD:RPAD + N]
    inv_deg = (1.0 / jnp.maximum(deg, 1.0)).reshape(N, 1)

    h1 = pl.pallas_call(
        _tc1_body,
        out_shape=jax.ShapeDtypeStruct((N, D), jnp.float32),
    )(acc1, inv_deg, x, W_l1, b_l1.reshape(1, D), W_r1,
      bn1_g.reshape(1, D), bn1_b.reshape(1, D))

    (acc2,) = _sc_agg(ch0, ch1, False)(h1, src_p, dst_p, z2, z1)

    C = W_fc.shape[0]
    out = pl.pallas_call(
        _tc2_body,
        out_shape=jax.ShapeDtypeStruct((N, C), jnp.float32),
    )(acc2, inv_deg, h1, W_l2, b_l2.reshape(1, D), W_r2,
      bn2_g.reshape(1, D), bn2_b.reshape(1, D), W_fc, b_fc.reshape(1, C))
    return out


# speedup vs baseline: 1.0218x; 1.0218x over previous
import jax
import jax.numpy as jnp
from jax import lax
from jax.experimental import pallas as pl
from jax.experimental.pallas import tpu as pltpu
from jax.experimental.pallas import tpu_sc as plsc

N = 10000
D = 128
NC = 2
NS = 16
NW = NC * NS
K = 128
RPT = 632
RPAD = NS * RPT
TRASH = N
SPLIT0 = 0.68


def _core_chunks(E):
    cht = -(-E // (NS * K))
    ch0 = 2 * round(SPLIT0 * cht / 2)
    ch0 = max(2, min(ch0, cht))
    ch1 = max(2, 2 * (-(-(cht - ch0) // 2)))
    return ch0, ch1


def _sc_agg(ch0, ch1, with_deg):
    chmax = max(ch0, ch1)
    mesh = plsc.VectorSubcoreMesh(core_axis_name="c", subcore_axis_name="s")

    out_type = [jax.ShapeDtypeStruct((NC, RPAD, D), jnp.float32)]
    scratch = [
        pltpu.VMEM((chmax, K), jnp.int32),
        pltpu.VMEM((K,), jnp.int32),
        pltpu.VMEM((K,), jnp.int32),
        pltpu.VMEM((K, D), jnp.float32),
        pltpu.VMEM((K, D), jnp.float32),
        pltpu.VMEM_SHARED((RPAD, D), jnp.float32),
        pltpu.SemaphoreType.DMA,
        pltpu.SemaphoreType.DMA,
        pltpu.SemaphoreType.DMA,
        pltpu.SemaphoreType.DMA,
    ]
    if with_deg:
        out_type.append(jax.ShapeDtypeStruct((NC * RPAD,), jnp.float32))
        scratch.append(pltpu.VMEM((K,), jnp.float32))
        scratch.append(pltpu.VMEM_SHARED((RPAD,), jnp.float32))
        scratch.append(pltpu.VMEM((RPT,), jnp.float32))

    def body(x_hbm, src_hbm, dst_hbm, z2_hbm, z1_hbm, *rest):
        if with_deg:
            (acc_out, deg_out, dst_v, srcb0, srcb1, rows0, rows1, acc_s,
             gsem0, gsem1, isem0, isem1, ones_v, deg_s, deg_stage) = rest
        else:
            (acc_out, dst_v, srcb0, srcb1, rows0, rows1, acc_s,
             gsem0, gsem1, isem0, isem1) = rest
        c = lax.axis_index("c")
        s = lax.axis_index("s")
        wid = s * NC + c
        base0 = wid * (chmax * K)
        my_chunks = jnp.where(c == 0, ch0, ch1)

        srcb = (srcb0, srcb1)
        rows = (rows0, rows1)
        gsem = (gsem0, gsem1)
        isem = (isem0, isem1)

        pltpu.sync_copy(dst_hbm.at[wid], dst_v)

        pltpu.sync_copy(z2_hbm.at[pl.ds(s * RPT, RPT)],
                        acc_s.at[pl.ds(s * RPT, RPT)])
        if with_deg:
            pltpu.sync_copy(z1_hbm.at[pl.ds(s * RPT, RPT)], deg_stage)
            pltpu.sync_copy(deg_stage, deg_s.at[pl.ds(s * RPT, RPT)])
            for j in range(K // 16):
                ones_v[pl.ds(j * 16, 16)] = jnp.ones((16,), jnp.float32)
        plsc.subcore_barrier()

        def src_start(i, b):
            pltpu.async_copy(src_hbm.at[pl.ds(base0 + i * K, K)],
                             srcb[b], isem[b])

        def src_wait(b):
            pltpu.make_async_copy(src_hbm.at[pl.ds(0, K)], srcb[b],
                                  isem[b]).wait()

        def gather_start(b):
            pltpu.async_copy(x_hbm.at[srcb[b]], rows[b], gsem[b])

        def gather_wait(b):
            pltpu.make_async_copy(x_hbm.at[srcb[b]], rows[b],
                                  gsem[b]).wait()

        def scatter(i, b):
            pltpu.sync_copy(rows[b], acc_s.at[dst_v.at[i]], add=True)
            if with_deg:
                pltpu.sync_copy(ones_v, deg_s.at[dst_v.at[i]], add=True)

        src_start(0, 0)
        src_wait(0)
        gather_start(0)
        src_start(1, 1)
        src_wait(1)
        gather_start(1)

        def pair(j, carry):
            i0 = 2 * j
            gather_wait(0)
            scatter(i0, 0)

            @pl.when(i0 + 2 < my_chunks)
            def _():
                src_start(i0 + 2, 0)
                src_wait(0)
                gather_start(0)

            gather_wait(1)
            scatter(i0 + 1, 1)

            @pl.when(i0 + 3 < my_chunks)
            def _():
                src_start(i0 + 3, 1)
                src_wait(1)
                gather_start(1)

            return carry

        lax.fori_loop(0, my_chunks // 2, pair, 0)
        plsc.subcore_barrier()

        pltpu.sync_copy(acc_s.at[pl.ds(s * RPT, RPT)],
                        acc_out.at[c, pl.ds(s * RPT, RPT)])
        if with_deg:
            pltpu.sync_copy(deg_s.at[pl.ds(s * RPT, RPT)], deg_stage)
            pltpu.sync_copy(deg_stage,
                            deg_out.at[pl.ds(c * RPAD + s * RPT, RPT)])

    return pl.kernel(body, out_type=out_type, mesh=mesh,
                     scratch_types=scratch)


def _split_edges(idx, E, ch0, ch1, fill):
    chmax = max(ch0, ch1)
    cap0 = NS * ch0 * K
    cap1 = NS * ch1 * K
    idx_p = jnp.concatenate(
        [idx, jnp.full((cap0 + cap1 - E,), fill, jnp.int32)])
    p0 = idx_p[:cap0].reshape(NS, ch0, K)
    p1 = idx_p[cap0:].reshape(NS, ch1, K)
    p0 = jnp.pad(p0, ((0, 0), (0, chmax - ch0), (0, 0)),
                 constant_values=fill)
    p1 = jnp.pad(p1, ((0, 0), (0, chmax - ch1), (0, 0)),
                 constant_values=fill)
    return jnp.stack([p0, p1], axis=1).reshape(NW, chmax, K)


def _dot_t(a, b):
    return lax.dot_general(a, b, (((1,), (1,)), ((), ())),
                           preferred_element_type=jnp.float32)


def _tc1_body(acc_ref, invd_ref, x_ref, wl_ref, bl_ref, wr_ref,
              g_ref, b_ref, out_ref):
    aggsum = acc_ref[0, :N, :] + acc_ref[1, :N, :]
    agg = aggsum * invd_ref[...]
    p = _dot_t(agg, wl_ref[...]) + bl_ref[...] + _dot_t(x_ref[...], wr_ref[...])
    mu = jnp.mean(p, axis=0, keepdims=True)
    var = jnp.mean((p - mu) ** 2, axis=0, keepdims=True)
    h = (p - mu) * lax.rsqrt(var + 1e-5) * g_ref[...] + b_ref[...]
    out_ref[...] = jnp.maximum(h, 0.0)


def _tc2_body(acc_ref, invd_ref, h_ref, wl_ref, bl_ref, wr_ref,
              g_ref, b_ref, wfc_ref, bfc_ref, out_ref):
    aggsum = acc_ref[0, :N, :] + acc_ref[1, :N, :]
    agg = aggsum * invd_ref[...]
    p = _dot_t(agg, wl_ref[...]) + bl_ref[...] + _dot_t(h_ref[...], wr_ref[...])
    mu = jnp.mean(p, axis=0, keepdims=True)
    var = jnp.mean((p - mu) ** 2, axis=0, keepdims=True)
    h2 = (p - mu) * lax.rsqrt(var + 1e-5) * g_ref[...] + b_ref[...]
    h2 = jnp.maximum(h2, 0.0)
    out_ref[...] = _dot_t(h2, wfc_ref[...]) + bfc_ref[...]


def kernel(x, edge_index, W_l1, b_l1, W_r1, bn1_g, bn1_b,
           W_l2, b_l2, W_r2, bn2_g, bn2_b, W_fc, b_fc):
    E = edge_index.shape[1]
    ch0, ch1 = _core_chunks(E)
    src_p = _split_edges(edge_index[0], E, ch0, ch1, 0).reshape(-1)
    dst_p = _split_edges(edge_index[1], E, ch0, ch1, TRASH)
    z2 = jnp.zeros((RPAD, D), jnp.float32)
    z1 = jnp.zeros((RPAD,), jnp.float32)

    acc1, degp = _sc_agg(ch0, ch1, True)(x, src_p, dst_p, z2, z1)
    deg = degp[:N] + degp[RPAD:RPAD + N]
    inv_deg = (1.0 / jnp.maximum(deg, 1.0)).reshape(N, 1)

    h1 = pl.pallas_call(
        _tc1_body,
        out_shape=jax.ShapeDtypeStruct((N, D), jnp.float32),
    )(acc1, inv_deg, x, W_l1, b_l1.reshape(1, D), W_r1,
      bn1_g.reshape(1, D), bn1_b.reshape(1, D))

    (acc2,) = _sc_agg(ch0, ch1, False)(h1, src_p, dst_p, z2, z1)

    C = W_fc.shape[0]
    out = pl.pallas_call(
        _tc2_body,
        out_shape=jax.ShapeDtypeStruct((N, C), jnp.float32),
    )(acc2, inv_deg, h1, W_l2, b_l2.reshape(1, D), W_r2,
      bn2_g.reshape(1, D), bn2_b.reshape(1, D), W_fc, b_fc.reshape(1, C))
    return out


# speedup vs baseline: 1.0866x; 1.0635x over previous
import jax
import jax.numpy as jnp
from jax import lax
from jax.experimental import pallas as pl
from jax.experimental.pallas import tpu as pltpu
from jax.experimental.pallas import tpu_sc as plsc

N = 10000
D = 128
NC = 2
NS = 16
NW = NC * NS
K = 128
RPT = 632
RPAD = NS * RPT
TRASH = N
SPLIT0 = 0.66


def _core_chunks(E):
    cht = -(-E // (NS * K))
    ch0 = 2 * round(SPLIT0 * cht / 2)
    ch0 = max(2, min(ch0, cht))
    ch1 = max(2, 2 * (-(-(cht - ch0) // 2)))
    return ch0, ch1


def _sc_agg(ch0, ch1, with_deg):
    chmax = max(ch0, ch1)
    mesh = plsc.VectorSubcoreMesh(core_axis_name="c", subcore_axis_name="s")

    out_type = [jax.ShapeDtypeStruct((NC, RPAD, D), jnp.float32)]
    scratch = [
        pltpu.VMEM((chmax, K), jnp.int32),
        pltpu.VMEM((K,), jnp.int32),
        pltpu.VMEM((K,), jnp.int32),
        pltpu.VMEM((K, D), jnp.float32),
        pltpu.VMEM((K, D), jnp.float32),
        pltpu.VMEM_SHARED((RPAD, D), jnp.float32),
        pltpu.SemaphoreType.DMA,
        pltpu.SemaphoreType.DMA,
        pltpu.SemaphoreType.DMA,
        pltpu.SemaphoreType.DMA,
    ]
    if with_deg:
        out_type.append(jax.ShapeDtypeStruct((NC * RPAD,), jnp.float32))
        scratch.append(pltpu.VMEM((K,), jnp.float32))
        scratch.append(pltpu.VMEM_SHARED((RPAD,), jnp.float32))
        scratch.append(pltpu.VMEM((RPT,), jnp.float32))

    def body(x_hbm, src_hbm, dst_hbm, z2_hbm, z1_hbm, *rest):
        if with_deg:
            (acc_out, deg_out, dst_v, srcb0, srcb1, rows0, rows1, acc_s,
             gsem0, gsem1, isem0, isem1, ones_v, deg_s, deg_stage) = rest
        else:
            (acc_out, dst_v, srcb0, srcb1, rows0, rows1, acc_s,
             gsem0, gsem1, isem0, isem1) = rest
        c = lax.axis_index("c")
        s = lax.axis_index("s")
        wid = s * NC + c
        base0 = wid * (chmax * K)
        my_chunks = jnp.where(c == 0, ch0, ch1)

        srcb = (srcb0, srcb1)
        rows = (rows0, rows1)
        gsem = (gsem0, gsem1)
        isem = (isem0, isem1)

        pltpu.sync_copy(dst_hbm.at[wid], dst_v)

        pltpu.sync_copy(z2_hbm.at[pl.ds(s * RPT, RPT)],
                        acc_s.at[pl.ds(s * RPT, RPT)])
        if with_deg:
            pltpu.sync_copy(z1_hbm.at[pl.ds(s * RPT, RPT)], deg_stage)
            pltpu.sync_copy(deg_stage, deg_s.at[pl.ds(s * RPT, RPT)])
            for j in range(K // 16):
                ones_v[pl.ds(j * 16, 16)] = jnp.ones((16,), jnp.float32)
        plsc.subcore_barrier()

        def src_start(i, b):
            pltpu.async_copy(src_hbm.at[pl.ds(base0 + i * K, K)],
                             srcb[b], isem[b])

        def src_wait(b):
            pltpu.make_async_copy(src_hbm.at[pl.ds(0, K)], srcb[b],
                                  isem[b]).wait()

        def gather_start(b):
            pltpu.async_copy(x_hbm.at[srcb[b]], rows[b], gsem[b])

        def gather_wait(b):
            pltpu.make_async_copy(x_hbm.at[srcb[b]], rows[b],
                                  gsem[b]).wait()

        def scatter(i, b):
            pltpu.sync_copy(rows[b], acc_s.at[dst_v.at[i]], add=True)
            if with_deg:
                pltpu.sync_copy(ones_v, deg_s.at[dst_v.at[i]], add=True)

        src_start(0, 0)
        src_wait(0)
        gather_start(0)
        src_start(1, 1)
        src_wait(1)
        gather_start(1)

        def pair(j, carry):
            i0 = 2 * j
            gather_wait(0)
            scatter(i0, 0)

            @pl.when(i0 + 2 < my_chunks)
            def _():
                src_start(i0 + 2, 0)
                src_wait(0)
                gather_start(0)

            gather_wait(1)
            scatter(i0 + 1, 1)

            @pl.when(i0 + 3 < my_chunks)
            def _():
                src_start(i0 + 3, 1)
                src_wait(1)
                gather_start(1)

            return carry

        lax.fori_loop(0, my_chunks // 2, pair, 0)
        plsc.subcore_barrier()

        pltpu.sync_copy(acc_s.at[pl.ds(s * RPT, RPT)],
                        acc_out.at[c, pl.ds(s * RPT, RPT)])
        if with_deg:
            pltpu.sync_copy(deg_s.at[pl.ds(s * RPT, RPT)], deg_stage)
            pltpu.sync_copy(deg_stage,
                            deg_out.at[pl.ds(c * RPAD + s * RPT, RPT)])

    return pl.kernel(body, out_type=out_type, mesh=mesh,
                     scratch_types=scratch)


def _split_edges(idx, E, ch0, ch1, fill):
    chmax = max(ch0, ch1)
    cap0 = NS * ch0 * K
    cap1 = NS * ch1 * K
    idx_p = jnp.concatenate(
        [idx, jnp.full((cap0 + cap1 - E,), fill, jnp.int32)])
    p0 = idx_p[:cap0].reshape(NS, ch0, K)
    p1 = idx_p[cap0:].reshape(NS, ch1, K)
    p0 = jnp.pad(p0, ((0, 0), (0, chmax - ch0), (0, 0)),
                 constant_values=fill)
    p1 = jnp.pad(p1, ((0, 0), (0, chmax - ch1), (0, 0)),
                 constant_values=fill)
    return jnp.stack([p0, p1], axis=1).reshape(NW, chmax, K)


def _dot_t(a, b):
    return lax.dot_general(a, b, (((1,), (1,)), ((), ())),
                           preferred_element_type=jnp.float32)


def _tc1_body(acc_ref, invd_ref, x_ref, wl_ref, bl_ref, wr_ref,
              g_ref, b_ref, out_ref):
    aggsum = acc_ref[0, :N, :] + acc_ref[1, :N, :]
    agg = aggsum * invd_ref[...]
    p = _dot_t(agg, wl_ref[...]) + bl_ref[...] + _dot_t(x_ref[...], wr_ref[...])
    mu = jnp.mean(p, axis=0, keepdims=True)
    var = jnp.mean((p - mu) ** 2, axis=0, keepdims=True)
    h = (p - mu) * lax.rsqrt(var + 1e-5) * g_ref[...] + b_ref[...]
    out_ref[...] = jnp.maximum(h, 0.0)


def _tc2_body(acc_ref, invd_ref, h_ref, wl_ref, bl_ref, wr_ref,
              g_ref, b_ref, wfc_ref, bfc_ref, out_ref):
    aggsum = acc_ref[0, :N, :] + acc_ref[1, :N, :]
    agg = aggsum * invd_ref[...]
    p = _dot_t(agg, wl_ref[...]) + bl_ref[...] + _dot_t(h_ref[...], wr_ref[...])
    mu = jnp.mean(p, axis=0, keepdims=True)
    var = jnp.mean((p - mu) ** 2, axis=0, keepdims=True)
    h2 = (p - mu) * lax.rsqrt(var + 1e-5) * g_ref[...] + b_ref[...]
    h2 = jnp.maximum(h2, 0.0)
    out_ref[...] = _dot_t(h2, wfc_ref[...]) + bfc_ref[...]


def kernel(x, edge_index, W_l1, b_l1, W_r1, bn1_g, bn1_b,
           W_l2, b_l2, W_r2, bn2_g, bn2_b, W_fc, b_fc):
    E = edge_index.shape[1]
    ch0, ch1 = _core_chunks(E)
    src_p = _split_edges(edge_index[0], E, ch0, ch1, 0).reshape(-1)
    dst_p = _split_edges(edge_index[1], E, ch0, ch1, TRASH)
    z2 = jnp.zeros((RPAD, D), jnp.float32)
    z1 = jnp.zeros((RPAD,), jnp.float32)

    acc1, degp = _sc_agg(ch0, ch1, True)(x, src_p, dst_p, z2, z1)
    deg = degp[:N] + degp[RPAD:RPAD + N]
    inv_deg = (1.0 / jnp.maximum(deg, 1.0)).reshape(N, 1)

    h1 = pl.pallas_call(
        _tc1_body,
        out_shape=jax.ShapeDtypeStruct((N, D), jnp.float32),
    )(acc1, inv_deg, x, W_l1, b_l1.reshape(1, D), W_r1,
      bn1_g.reshape(1, D), bn1_b.reshape(1, D))

    (acc2,) = _sc_agg(ch0, ch1, False)(h1, src_p, dst_p, z2, z1)

    C = W_fc.shape[0]
    out = pl.pallas_call(
        _tc2_body,
        out_shape=jax.ShapeDtypeStruct((N, C), jnp.float32),
    )(acc2, inv_deg, h1, W_l2, b_l2.reshape(1, D), W_r2,
      bn2_g.reshape(1, D), bn2_b.reshape(1, D), W_fc, b_fc.reshape(1, C))
    return out
